# Initial kernel scaffold; baseline (speedup 1.0000x reference)
import jax
import jax.numpy as jnp
from jax.experimental import pallas as pl


def kernel(x, W_in, b_in, tod_table, dow_table, adp):
    raise NotImplementedError("write your pallas kernel here")


# speedup vs baseline: 2.5318x; 2.5318x over previous
import jax
import jax.numpy as jnp
from jax import lax
from jax.experimental import pallas as pl
from jax.experimental.pallas import tpu as pltpu
from jax.experimental.pallas import tpu_sc as plsc

B, L, N, C = 32, 12, 2000, 4
P = B * L * N
LN = L * N
DW = 32
STEP_PER_DAY = 23
DAY_PER_WEEK = 6

NW = 32
PPW = P // NW
CHUNK = 480
GROUPS = CHUNK // 16
NCHUNKS = PPW // CHUNK


def _sc_embed(x_hbm, wb_hbm, tod_hbm, dow_hbm, adpt_hbm, out_hbm,
              x_v, adp_v, out_v, wb_v, tod_v, dow_v):
    nc = 2
    wid = lax.axis_index("s") * nc + lax.axis_index("c")

    pltpu.sync_copy(wb_hbm, wb_v)
    pltpu.sync_copy(tod_hbm, tod_v)
    pltpu.sync_copy(dow_hbm, dow_v)

    iota = lax.iota(jnp.int32, 16)
    zeros = jnp.zeros((16,), jnp.int32)
    w_spl = [plsc.load_gather(wb_v, [zeros + (8 + c)]) for c in range(8)]
    b_spl = [plsc.load_gather(wb_v, [zeros + (16 + c)]) for c in range(8)]

    base0 = wid * PPW

    def chunk_body(ci, carry):
        base = base0 + ci * CHUNK
        pltpu.sync_copy(x_hbm.at[pl.ds(base * C, CHUNK * C)], x_v)
        pltpu.sync_copy(adpt_hbm.at[ci], adp_v)

        def group_body(g, inner):
            pids = g * 16 + iota
            xb = pids * C
            x0 = plsc.load_gather(x_v, [xb])
            x2 = plsc.load_gather(x_v, [xb + 2])
            x3 = plsc.load_gather(x_v, [xb + 3])
            ti = (x3 * float(STEP_PER_DAY)).astype(jnp.int32) * 8
            di = (x2 * float(DAY_PER_WEEK)).astype(jnp.int32) * 8
            ob = pids * DW
            for c in range(8):
                col = x0 * w_spl[c] + b_spl[c]
                plsc.store_scatter(out_v, [ob + c], col)
            for c in range(8):
                v = plsc.load_gather(tod_v, [ti + c])
                plsc.store_scatter(out_v, [ob + (8 + c)], v)
            for c in range(8):
                v = plsc.load_gather(dow_v, [di + c])
                plsc.store_scatter(out_v, [ob + (16 + c)], v)
            for c in range(8):
                v = adp_v[pl.ds(c * CHUNK + g * 16, 16)]
                plsc.store_scatter(out_v, [ob + (24 + c)], v)
            return inner

        lax.fori_loop(0, GROUPS, group_body, 0)
        pltpu.sync_copy(out_v, out_hbm.at[pl.ds(base * DW, CHUNK * DW)])
        return carry

    lax.fori_loop(0, NCHUNKS, chunk_body, 0)


@jax.jit
def _run(x_flat, wb, tod_flat, dow_flat, adp_t):
    fn = pl.kernel(
        _sc_embed,
        out_type=jax.ShapeDtypeStruct((P * DW,), jnp.float32),
        mesh=plsc.VectorSubcoreMesh(core_axis_name="c", subcore_axis_name="s"),
        compiler_params=pltpu.CompilerParams(needs_layout_passes=False),
        scratch_types=[
            pltpu.VMEM((CHUNK * C,), jnp.float32),
            pltpu.VMEM((8 * CHUNK,), jnp.float32),
            pltpu.VMEM((CHUNK * DW,), jnp.float32),
            pltpu.VMEM((24,), jnp.float32),
            pltpu.VMEM((24 * 8,), jnp.float32),
            pltpu.VMEM((8 * 8,), jnp.float32),
        ],
    )
    return fn(x_flat, wb, tod_flat, dow_flat, adp_t)


def kernel(x, W_in, b_in, tod_table, dow_table, adp):
    x_flat = x.reshape(P * C)
    wb = jnp.concatenate([jnp.zeros((8,), jnp.float32), W_in[0], b_in])
    dow_pad = jnp.zeros((8, 8), jnp.float32).at[:7].set(dow_table).reshape(64)
    adp_t = (adp.reshape(LN, 8).T.reshape(8, NCHUNKS, CHUNK)
             .transpose(1, 0, 2).reshape(NCHUNKS, 8 * CHUNK))
    out = _run(x_flat, wb, tod_table.reshape(24 * 8), dow_pad, adp_t)
    return out.reshape(B, L, N, DW)


# speedup vs baseline: 3.9491x; 1.5598x over previous
import jax
import jax.numpy as jnp
from jax import lax
from jax.experimental import pallas as pl
from jax.experimental.pallas import tpu as pltpu
from jax.experimental.pallas import tpu_sc as plsc

B, L, N, C = 32, 12, 2000, 4
LN = L * N
DW = 32
STEP_PER_DAY = 23
DAY_PER_WEEK = 6

NW = 32
CHUNK = 400
NCH = N // CHUNK
GROUPS = CHUNK // 16


def _sc_embed(x_hbm, wb_hbm, tod_hbm, dow_hbm, adpt_hbm, out_hbm,
              x_v, adp_v, out_v, wb_v, tod_v, dow_v):
    nc = 2
    wid = lax.axis_index("s") * nc + lax.axis_index("c")

    pltpu.sync_copy(wb_hbm, wb_v)
    pltpu.sync_copy(tod_hbm, tod_v)
    pltpu.sync_copy(dow_hbm, dow_v)

    iota = lax.iota(jnp.int32, 16)
    zeros = jnp.zeros((16,), jnp.int32)
    w_spl = [plsc.load_gather(wb_v, [zeros + (8 + c)]) for c in range(8)]
    b_spl = [plsc.load_gather(wb_v, [zeros + (16 + c)]) for c in range(8)]

    def l_body(ci, carry):
        li = ci // NCH
        n0 = (ci % NCH) * CHUNK
        pltpu.sync_copy(x_hbm.at[wid, li, pl.ds(n0, CHUNK)], x_v)
        pltpu.sync_copy(adpt_hbm.at[ci], adp_v)

        def group_body(g, inner):
            pids = g * 16 + iota
            x0 = plsc.load_gather(x_v, [pids, zeros])
            x2 = plsc.load_gather(x_v, [pids, zeros + 2])
            x3 = plsc.load_gather(x_v, [pids, zeros + 3])
            ti = (x3 * float(STEP_PER_DAY)).astype(jnp.int32) * 8
            di = (x2 * float(DAY_PER_WEEK)).astype(jnp.int32) * 8
            for c in range(8):
                col = x0 * w_spl[c] + b_spl[c]
                plsc.store_scatter(out_v, [pids, zeros + c], col)
            for c in range(8):
                v = plsc.load_gather(tod_v, [ti + c])
                plsc.store_scatter(out_v, [pids, zeros + (8 + c)], v)
            for c in range(8):
                v = plsc.load_gather(dow_v, [di + c])
                plsc.store_scatter(out_v, [pids, zeros + (16 + c)], v)
            for c in range(8):
                v = adp_v[pl.ds(c * CHUNK + g * 16, 16)]
                plsc.store_scatter(out_v, [pids, zeros + (24 + c)], v)
            return inner

        lax.fori_loop(0, GROUPS, group_body, 0)
        pltpu.sync_copy(out_v, out_hbm.at[wid, li, pl.ds(n0, CHUNK)])
        return carry

    lax.fori_loop(0, L * NCH, l_body, 0)


@jax.jit
def _run(x, wb, tod_flat, dow_flat, adp_t):
    fn = pl.kernel(
        _sc_embed,
        out_type=jax.ShapeDtypeStruct((B, L, N, DW), jnp.float32),
        mesh=plsc.VectorSubcoreMesh(core_axis_name="c", subcore_axis_name="s"),
        compiler_params=pltpu.CompilerParams(needs_layout_passes=False),
        scratch_types=[
            pltpu.VMEM((CHUNK, C), jnp.float32),
            pltpu.VMEM((8 * CHUNK,), jnp.float32),
            pltpu.VMEM((CHUNK, DW), jnp.float32),
            pltpu.VMEM((24,), jnp.float32),
            pltpu.VMEM((24 * 8,), jnp.float32),
            pltpu.VMEM((8 * 8,), jnp.float32),
        ],
    )
    return fn(x, wb, tod_flat, dow_flat, adp_t)


def kernel(x, W_in, b_in, tod_table, dow_table, adp):
    wb = jnp.concatenate([jnp.zeros((8,), jnp.float32), W_in[0], b_in])
    dow_pad = jnp.zeros((8, 8), jnp.float32).at[:7].set(dow_table).reshape(64)
    adp_t = (adp.transpose(0, 2, 1).reshape(L, 8, NCH, CHUNK)
             .transpose(0, 2, 1, 3).reshape(L * NCH, 8 * CHUNK))
    return _run(x, wb, tod_table.reshape(24 * 8), dow_pad, adp_t)


# speedup vs baseline: 4.3971x; 1.1134x over previous
import jax
import jax.numpy as jnp
from jax import lax
from jax.experimental import pallas as pl
from jax.experimental.pallas import tpu as pltpu
from jax.experimental.pallas import tpu_sc as plsc

B, L, N, C = 32, 12, 2000, 4
LN = L * N
DW = 32
STEP_PER_DAY = 23
DAY_PER_WEEK = 6

NW = 32
CHUNK = 400
NCH = N // CHUNK
GROUPS = CHUNK // 16


def _sc_embed(x_hbm, wb_hbm, tod_hbm, dow_hbm, adpt_hbm, out_hbm,
              x_v, adp_v, out_v, wb_v, tod_v, dow_v):
    nc = 2
    wid = lax.axis_index("s") * nc + lax.axis_index("c")

    pltpu.sync_copy(wb_hbm, wb_v)
    pltpu.sync_copy(tod_hbm, tod_v)
    pltpu.sync_copy(dow_hbm, dow_v)

    iota = lax.iota(jnp.int32, 16)
    zeros = jnp.zeros((16,), jnp.int32)
    w_spl = [plsc.load_gather(wb_v, [zeros + (8 + c)]) for c in range(8)]
    b_spl = [plsc.load_gather(wb_v, [zeros + (16 + c)]) for c in range(8)]

    def l_body(ci, carry):
        li = ci // NCH
        n0 = (ci % NCH) * CHUNK
        pltpu.sync_copy(x_hbm.at[wid, li, pl.ds(n0, CHUNK)], x_v)
        pltpu.sync_copy(adpt_hbm.at[ci], adp_v)

        @plsc.parallel_loop(0, GROUPS, unroll=4)
        def group_body(g):
            pids = g * 16 + iota
            x0 = plsc.load_gather(x_v, [pids, zeros])
            x2 = plsc.load_gather(x_v, [pids, zeros + 2])
            x3 = plsc.load_gather(x_v, [pids, zeros + 3])
            ti = (x3 * float(STEP_PER_DAY)).astype(jnp.int32) * 8
            di = (x2 * float(DAY_PER_WEEK)).astype(jnp.int32) * 8
            for c in range(8):
                col = x0 * w_spl[c] + b_spl[c]
                plsc.store_scatter(out_v, [pids, zeros + c], col)
            for c in range(8):
                v = plsc.load_gather(tod_v, [ti + c])
                plsc.store_scatter(out_v, [pids, zeros + (8 + c)], v)
            for c in range(8):
                v = plsc.load_gather(dow_v, [di + c])
                plsc.store_scatter(out_v, [pids, zeros + (16 + c)], v)
            for c in range(8):
                v = adp_v[pl.ds(c * CHUNK + g * 16, 16)]
                plsc.store_scatter(out_v, [pids, zeros + (24 + c)], v)

        pltpu.sync_copy(out_v, out_hbm.at[wid, li, pl.ds(n0, CHUNK)])
        return carry

    lax.fori_loop(0, L * NCH, l_body, 0)


@jax.jit
def _run(x, wb, tod_flat, dow_flat, adp_t):
    fn = pl.kernel(
        _sc_embed,
        out_type=jax.ShapeDtypeStruct((B, L, N, DW), jnp.float32),
        mesh=plsc.VectorSubcoreMesh(core_axis_name="c", subcore_axis_name="s"),
        compiler_params=pltpu.CompilerParams(needs_layout_passes=False),
        scratch_types=[
            pltpu.VMEM((CHUNK, C), jnp.float32),
            pltpu.VMEM((8 * CHUNK,), jnp.float32),
            pltpu.VMEM((CHUNK, DW), jnp.float32),
            pltpu.VMEM((24,), jnp.float32),
            pltpu.VMEM((24 * 8,), jnp.float32),
            pltpu.VMEM((8 * 8,), jnp.float32),
        ],
    )
    return fn(x, wb, tod_flat, dow_flat, adp_t)


def kernel(x, W_in, b_in, tod_table, dow_table, adp):
    wb = jnp.concatenate([jnp.zeros((8,), jnp.float32), W_in[0], b_in])
    dow_pad = jnp.zeros((8, 8), jnp.float32).at[:7].set(dow_table).reshape(64)
    adp_t = (adp.transpose(0, 2, 1).reshape(L, 8, NCH, CHUNK)
             .transpose(0, 2, 1, 3).reshape(L * NCH, 8 * CHUNK))
    return _run(x, wb, tod_table.reshape(24 * 8), dow_pad, adp_t)


# speedup vs baseline: 43.2462x; 9.8352x over previous
import jax
import jax.numpy as jnp
from jax import lax
from jax.experimental import pallas as pl
from jax.experimental.pallas import tpu as pltpu
from jax.experimental.pallas import tpu_sc as plsc

B, L, N, C = 32, 12, 2000, 4
DW = 32
STEP_PER_DAY = 23
DAY_PER_WEEK = 6

NW = 32
GROUPS = N // 16


def _sc_embed(x_hbm, wb_hbm, tod_hbm, dow_hbm, adpt_hbm, out_hbm,
              x_v, adp_v, buf_a, buf_b, wb_v, tod_v, dow_v):
    nc = 2
    wid = lax.axis_index("s") * nc + lax.axis_index("c")

    pltpu.sync_copy(wb_hbm, wb_v)
    pltpu.sync_copy(tod_hbm, tod_v)
    pltpu.sync_copy(dow_hbm, dow_v)

    iota = lax.iota(jnp.int32, 16)
    zeros = jnp.zeros((16,), jnp.int32)
    w_spl = [plsc.load_gather(wb_v, [zeros + (8 + c)]) for c in range(8)]
    b_spl = [plsc.load_gather(wb_v, [zeros + (16 + c)]) for c in range(8)]

    def l_body(li, carry):
        pltpu.sync_copy(x_hbm.at[wid, li], x_v)
        pltpu.sync_copy(adpt_hbm.at[li], adp_v)

        @plsc.parallel_loop(0, GROUPS, unroll=4)
        def proj_body(g):
            x0 = x_v[0, pl.ds(g * 16, 16)]
            for c in range(8):
                buf_a[c, pl.ds(g * 16, 16)] = x0 * w_spl[c] + b_spl[c]

        pltpu.sync_copy(buf_a, out_hbm.at[wid, li, pl.ds(0, 8)])

        @plsc.parallel_loop(0, GROUPS, unroll=4)
        def tod_body(g):
            x3 = x_v[3, pl.ds(g * 16, 16)]
            ti = (x3 * float(STEP_PER_DAY)).astype(jnp.int32)
            for c in range(8):
                buf_b[c, pl.ds(g * 16, 16)] = plsc.load_gather(
                    tod_v, [zeros + c, ti])

        pltpu.sync_copy(buf_b, out_hbm.at[wid, li, pl.ds(8, 8)])

        @plsc.parallel_loop(0, GROUPS, unroll=4)
        def dow_body(g):
            x2 = x_v[2, pl.ds(g * 16, 16)]
            di = (x2 * float(DAY_PER_WEEK)).astype(jnp.int32)
            for c in range(8):
                buf_a[c, pl.ds(g * 16, 16)] = plsc.load_gather(
                    dow_v, [zeros + c, di])

        pltpu.sync_copy(buf_a, out_hbm.at[wid, li, pl.ds(16, 8)])
        pltpu.sync_copy(adp_v, out_hbm.at[wid, li, pl.ds(24, 8)])
        return carry

    lax.fori_loop(0, L, l_body, 0)


@jax.jit
def _run(x_t, wb, tod_t, dow_t, adp_t):
    fn = pl.kernel(
        _sc_embed,
        out_type=jax.ShapeDtypeStruct((B, L, DW, N), jnp.float32),
        mesh=plsc.VectorSubcoreMesh(core_axis_name="c", subcore_axis_name="s"),
        compiler_params=pltpu.CompilerParams(needs_layout_passes=False),
        scratch_types=[
            pltpu.VMEM((C, N), jnp.float32),
            pltpu.VMEM((8, N), jnp.float32),
            pltpu.VMEM((8, N), jnp.float32),
            pltpu.VMEM((8, N), jnp.float32),
            pltpu.VMEM((24,), jnp.float32),
            pltpu.VMEM((8, 24), jnp.float32),
            pltpu.VMEM((8, 8), jnp.float32),
        ],
    )
    return fn(x_t, wb, tod_t, dow_t, adp_t)


def kernel(x, W_in, b_in, tod_table, dow_table, adp):
    x_t = x.transpose(0, 1, 3, 2)
    adp_t = adp.transpose(0, 2, 1)
    wb = jnp.concatenate([jnp.zeros((8,), jnp.float32), W_in[0], b_in])
    tod_t = tod_table.T
    dow_t = jnp.zeros((8, 8), jnp.float32).at[:, :7].set(dow_table.T)
    out_t = _run(x_t, wb, tod_t, dow_t, adp_t)
    return out_t.transpose(0, 1, 3, 2)


# speedup vs baseline: 56.0884x; 1.2970x over previous
import jax
import jax.numpy as jnp
from jax import lax
from jax.experimental import pallas as pl
from jax.experimental.pallas import tpu as pltpu
from jax.experimental.pallas import tpu_sc as plsc

B, L, N, C = 32, 12, 2000, 4
DW = 32
STEP_PER_DAY = 23
DAY_PER_WEEK = 6

NW = 32
GROUPS = N // 16


def _sc_embed(x_hbm, wb_hbm, tod_hbm, dow_hbm, adpt_hbm, out_hbm,
              x_v0, x_v1, adp_v, buf_a, buf_b, buf_c, wb_v, tod_v, dow_v,
              sem_x0, sem_x1, sem_ai, sem_ao, sem_a, sem_b, sem_c):
    nc = 2
    wid = lax.axis_index("s") * nc + lax.axis_index("c")

    pltpu.sync_copy(wb_hbm, wb_v)
    pltpu.sync_copy(tod_hbm, tod_v)
    pltpu.sync_copy(dow_hbm, dow_v)

    iota = lax.iota(jnp.int32, 16)
    zeros = jnp.zeros((16,), jnp.int32)
    w_spl = [plsc.load_gather(wb_v, [zeros + (8 + c)]) for c in range(8)]
    b_spl = [plsc.load_gather(wb_v, [zeros + (16 + c)]) for c in range(8)]

    def x_in(li, x_v, sem):
        return pltpu.make_async_copy(x_hbm.at[wid, li], x_v, sem)

    def adp_in(li):
        return pltpu.make_async_copy(adpt_hbm.at[li], adp_v, sem_ai)

    def sec_out(buf, li, c0, sem):
        return pltpu.make_async_copy(
            buf, out_hbm.at[wid, li, pl.ds(c0, 8)], sem)

    def proj_sec(x_v, li):
        @plsc.parallel_loop(0, GROUPS, unroll=4)
        def proj_body(g):
            x0 = x_v[0, pl.ds(g * 16, 16)]
            for c in range(8):
                buf_a[c, pl.ds(g * 16, 16)] = x0 * w_spl[c] + b_spl[c]
        sec_out(buf_a, li, 0, sem_a).start()

    def tod_sec(x_v, li):
        @plsc.parallel_loop(0, GROUPS, unroll=4)
        def tod_body(g):
            x3 = x_v[3, pl.ds(g * 16, 16)]
            ti = (x3 * float(STEP_PER_DAY)).astype(jnp.int32)
            for c in range(8):
                buf_b[c, pl.ds(g * 16, 16)] = plsc.load_gather(
                    tod_v, [zeros + c, ti])
        sec_out(buf_b, li, 8, sem_b).start()

    def dow_sec(x_v, li):
        @plsc.parallel_loop(0, GROUPS, unroll=4)
        def dow_body(g):
            x2 = x_v[2, pl.ds(g * 16, 16)]
            di = (x2 * float(DAY_PER_WEEK)).astype(jnp.int32)
            for c in range(8):
                buf_c[c, pl.ds(g * 16, 16)] = plsc.load_gather(
                    dow_v, [zeros + c, di])
        sec_out(buf_c, li, 16, sem_c).start()

    def adp_sec(li):
        adp_in(li).wait()
        pltpu.make_async_copy(
            adp_v, out_hbm.at[wid, li, pl.ds(24, 8)], sem_ao).start()

    x_in(0, x_v0, sem_x0).start()
    adp_in(0).start()

    def phase(li, x_v, sem_x):
        x_in(li, x_v, sem_x).wait()

        @pl.when(li > 0)
        def _():
            sec_out(buf_a, li, 0, sem_a).wait()
            sec_out(buf_b, li, 8, sem_b).wait()
            sec_out(buf_c, li, 16, sem_c).wait()

        proj_sec(x_v, li)
        tod_sec(x_v, li)
        dow_sec(x_v, li)
        adp_sec(li)

    def l_body(i, carry):
        l0 = 2 * i
        l1 = l0 + 1
        phase(l0, x_v0, sem_x0)
        x_in(l1, x_v1, sem_x1).start()
        pltpu.make_async_copy(
            adp_v, out_hbm.at[wid, l0, pl.ds(24, 8)], sem_ao).wait()
        adp_in(l1).start()

        phase(l1, x_v1, sem_x1)

        @pl.when(i < (L // 2) - 1)
        def _():
            x_in(l1 + 1, x_v0, sem_x0).start()
            pltpu.make_async_copy(
                adp_v, out_hbm.at[wid, l1, pl.ds(24, 8)], sem_ao).wait()
            adp_in(l1 + 1).start()
        return carry

    lax.fori_loop(0, L // 2, l_body, 0)

    sec_out(buf_a, L - 1, 0, sem_a).wait()
    sec_out(buf_b, L - 1, 8, sem_b).wait()
    sec_out(buf_c, L - 1, 16, sem_c).wait()
    pltpu.make_async_copy(
        adp_v, out_hbm.at[wid, L - 1, pl.ds(24, 8)], sem_ao).wait()


@jax.jit
def _run(x_t, wb, tod_t, dow_t, adp_t):
    fn = pl.kernel(
        _sc_embed,
        out_type=jax.ShapeDtypeStruct((B, L, DW, N), jnp.float32),
        mesh=plsc.VectorSubcoreMesh(core_axis_name="c", subcore_axis_name="s"),
        compiler_params=pltpu.CompilerParams(needs_layout_passes=False),
        scratch_types=[
            pltpu.VMEM((C, N), jnp.float32),
            pltpu.VMEM((C, N), jnp.float32),
            pltpu.VMEM((8, N), jnp.float32),
            pltpu.VMEM((8, N), jnp.float32),
            pltpu.VMEM((8, N), jnp.float32),
            pltpu.VMEM((8, N), jnp.float32),
            pltpu.VMEM((24,), jnp.float32),
            pltpu.VMEM((8, 24), jnp.float32),
            pltpu.VMEM((8, 8), jnp.float32),
            pltpu.SemaphoreType.DMA,
            pltpu.SemaphoreType.DMA,
            pltpu.SemaphoreType.DMA,
            pltpu.SemaphoreType.DMA,
            pltpu.SemaphoreType.DMA,
            pltpu.SemaphoreType.DMA,
            pltpu.SemaphoreType.DMA,
        ],
    )
    return fn(x_t, wb, tod_t, dow_t, adp_t)


def kernel(x, W_in, b_in, tod_table, dow_table, adp):
    x_t = x.transpose(0, 1, 3, 2)
    adp_t = adp.transpose(0, 2, 1)
    wb = jnp.concatenate([jnp.zeros((8,), jnp.float32), W_in[0], b_in])
    tod_t = tod_table.T
    dow_t = jnp.zeros((8, 8), jnp.float32).at[:, :7].set(dow_table.T)
    out_t = _run(x_t, wb, tod_t, dow_t, adp_t)
    return out_t.transpose(0, 1, 3, 2)


# speedup vs baseline: 74.3154x; 1.3250x over previous
import jax
import jax.numpy as jnp
from jax import lax
from jax.experimental import pallas as pl
from jax.experimental.pallas import tpu as pltpu
from jax.experimental.pallas import tpu_sc as plsc

B, L, N, C = 32, 12, 2000, 4
DW = 32
STEP_PER_DAY = 23
DAY_PER_WEEK = 6

NW = 32
GROUPS = N // 16


def _sc_embed(x_hbm, wb_hbm, tod_hbm, dow_hbm, adpt_hbm, out_hbm,
              x_v0, x_v1, adp_sh, buf_a, buf_b, buf_c, wb_v, tod_v, dow_v,
              sem_x0, sem_x1, sem_st, sem_ao, sem_a, sem_b, sem_c):
    nc = 2
    sid = lax.axis_index("s")
    wid = sid * nc + lax.axis_index("c")

    pltpu.make_async_copy(x_hbm.at[wid, 0], x_v0, sem_x0).start()

    @pl.when(sid < L)
    def _():
        cp = pltpu.make_async_copy(adpt_hbm.at[sid], adp_sh.at[sid], sem_st)
        cp.start()
        cp.wait()

    pltpu.sync_copy(wb_hbm, wb_v)
    pltpu.sync_copy(tod_hbm, tod_v)
    pltpu.sync_copy(dow_hbm, dow_v)
    plsc.subcore_barrier()

    iota = lax.iota(jnp.int32, 16)
    zeros = jnp.zeros((16,), jnp.int32)
    w_spl = [plsc.load_gather(wb_v, [zeros + (8 + c)]) for c in range(8)]
    b_spl = [plsc.load_gather(wb_v, [zeros + (16 + c)]) for c in range(8)]

    def x_in(li, x_v, sem):
        return pltpu.make_async_copy(x_hbm.at[wid, li], x_v, sem)

    def adp_out(li):
        return pltpu.make_async_copy(
            adp_sh.at[li], out_hbm.at[wid, li, pl.ds(24, 8)], sem_ao)

    def sec_out(buf, li, c0, sem):
        return pltpu.make_async_copy(
            buf, out_hbm.at[wid, li, pl.ds(c0, 8)], sem)

    def proj_sec(x_v, li):
        @plsc.parallel_loop(0, GROUPS, unroll=4)
        def proj_body(g):
            x0 = x_v[0, pl.ds(g * 16, 16)]
            for c in range(8):
                buf_a[c, pl.ds(g * 16, 16)] = x0 * w_spl[c] + b_spl[c]
        sec_out(buf_a, li, 0, sem_a).start()

    def tod_sec(x_v, li):
        @plsc.parallel_loop(0, GROUPS, unroll=4)
        def tod_body(g):
            x3 = x_v[3, pl.ds(g * 16, 16)]
            ti = (x3 * float(STEP_PER_DAY)).astype(jnp.int32)
            for c in range(8):
                buf_b[c, pl.ds(g * 16, 16)] = plsc.load_gather(
                    tod_v, [zeros + c, ti])
        sec_out(buf_b, li, 8, sem_b).start()

    def dow_sec(x_v, li):
        @plsc.parallel_loop(0, GROUPS, unroll=4)
        def dow_body(g):
            x2 = x_v[2, pl.ds(g * 16, 16)]
            di = (x2 * float(DAY_PER_WEEK)).astype(jnp.int32)
            for c in range(8):
                buf_c[c, pl.ds(g * 16, 16)] = plsc.load_gather(
                    dow_v, [zeros + c, di])
        sec_out(buf_c, li, 16, sem_c).start()

    def phase(li, x_v, sem_x):
        x_in(li, x_v, sem_x).wait()

        @pl.when(li > 0)
        def _():
            sec_out(buf_a, li, 0, sem_a).wait()
            sec_out(buf_b, li, 8, sem_b).wait()
            sec_out(buf_c, li, 16, sem_c).wait()
            adp_out(li).wait()

        proj_sec(x_v, li)
        tod_sec(x_v, li)
        dow_sec(x_v, li)
        adp_out(li).start()

    def l_body(i, carry):
        l0 = 2 * i
        l1 = l0 + 1
        phase(l0, x_v0, sem_x0)
        x_in(l1, x_v1, sem_x1).start()
        phase(l1, x_v1, sem_x1)

        @pl.when(i < (L // 2) - 1)
        def _():
            x_in(l1 + 1, x_v0, sem_x0).start()
        return carry

    lax.fori_loop(0, L // 2, l_body, 0)

    sec_out(buf_a, L - 1, 0, sem_a).wait()
    sec_out(buf_b, L - 1, 8, sem_b).wait()
    sec_out(buf_c, L - 1, 16, sem_c).wait()
    adp_out(L - 1).wait()


@jax.jit
def _run(x_t, wb, tod_t, dow_t, adp_t):
    fn = pl.kernel(
        _sc_embed,
        out_type=jax.ShapeDtypeStruct((B, L, DW, N), jnp.float32),
        mesh=plsc.VectorSubcoreMesh(core_axis_name="c", subcore_axis_name="s"),
        compiler_params=pltpu.CompilerParams(needs_layout_passes=False),
        scratch_types=[
            pltpu.VMEM((C, N), jnp.float32),
            pltpu.VMEM((C, N), jnp.float32),
            pltpu.VMEM_SHARED((L, 8, N), jnp.float32),
            pltpu.VMEM((8, N), jnp.float32),
            pltpu.VMEM((8, N), jnp.float32),
            pltpu.VMEM((8, N), jnp.float32),
            pltpu.VMEM((24,), jnp.float32),
            pltpu.VMEM((8, 24), jnp.float32),
            pltpu.VMEM((8, 8), jnp.float32),
            pltpu.SemaphoreType.DMA,
            pltpu.SemaphoreType.DMA,
            pltpu.SemaphoreType.DMA,
            pltpu.SemaphoreType.DMA,
            pltpu.SemaphoreType.DMA,
            pltpu.SemaphoreType.DMA,
            pltpu.SemaphoreType.DMA,
        ],
    )
    return fn(x_t, wb, tod_t, dow_t, adp_t)


def kernel(x, W_in, b_in, tod_table, dow_table, adp):
    x_t = x.transpose(0, 1, 3, 2)
    adp_t = adp.transpose(0, 2, 1)
    wb = jnp.concatenate([jnp.zeros((8,), jnp.float32), W_in[0], b_in])
    tod_t = tod_table.T
    dow_t = jnp.zeros((8, 8), jnp.float32).at[:, :7].set(dow_table.T)
    out_t = _run(x_t, wb, tod_t, dow_t, adp_t)
    return out_t.transpose(0, 1, 3, 2)
